# baseline (speedup 1.0000x reference)
import functools
import jax
import jax.numpy as jnp
from jax import lax
from jax.experimental import pallas as pl
from jax.experimental.pallas import tpu as pltpu
from jax.experimental.pallas import tpu_sc as plsc

N_NODES = 10000
N_EDGES = 320000
HIDDEN = 128
NUM_LAYERS = 3
N_GRAPHS = 64
N_PAIRS = 4096

NC = 2
NS = 16
NW = NC * NS

BS = 320
SC_ROWS = NS * BS
DUMP = SC_ROWS
BLK = 256
NBLK = N_EDGES // BLK
CCH = 80
NCH_C = N_EDGES // NW // CCH
LCH = 128
EPAD = N_EDGES + NW * 8 + LCH
PW = 16

_mesh = plsc.VectorSubcoreMesh(core_axis_name="c", subcore_axis_name="s")
_sc_params = pltpu.CompilerParams(use_tc_tiling_on_sc=False,
                                  needs_layout_passes=False)


def _dotd(a, b):
    return jnp.dot(a.astype(jnp.bfloat16), b.astype(jnp.bfloat16),
                   preferred_element_type=jnp.float32)


def _count_body(dst_ref, cnt_ref):
    @pl.when(pl.program_id(0) == 0)
    def _():
        cnt_ref[...] = jnp.zeros_like(cnt_ref)
    d = jnp.broadcast_to(jnp.reshape(dst_ref[...], (1, BLK)), (NW, BLK))
    lo = lax.broadcasted_iota(jnp.int32, (NW, BLK), 0) * BS
    oh = ((d >= lo) & (d < lo + BS)).astype(jnp.float32)
    cnt_ref[...] += jnp.sum(oh, axis=1, keepdims=True)


_count_call = pl.pallas_call(
    _count_body,
    grid=(NBLK,),
    in_specs=[pl.BlockSpec((1, 1, BLK), lambda i: (i, 0, 0))],
    out_specs=pl.BlockSpec((NW, 1), lambda i: (0, 0)),
    out_shape=jax.ShapeDtypeStruct((NW, 1), jnp.float32),
)


def _rank_body(dst_ref, cnt_ref, p_ref, meta_ref, carry, off):
    @pl.when(pl.program_id(0) == 0)
    def _():
        c = cnt_ref[...]
        c8 = jnp.floor((c + 7.0) * 0.125) * 8.0
        ltri = (lax.broadcasted_iota(jnp.int32, (NW, NW), 0)
                > lax.broadcasted_iota(jnp.int32, (NW, NW), 1)).astype(jnp.float32)
        off[...] = jnp.dot(ltri, c8, preferred_element_type=jnp.float32,
                           precision=lax.Precision.HIGHEST)
        carry[...] = jnp.zeros_like(carry)
        meta_ref[...] = jnp.concatenate([c, off[...]], axis=1).astype(jnp.int32)

    d = jnp.broadcast_to(jnp.reshape(dst_ref[...], (1, BLK)), (NW, BLK))
    lo = lax.broadcasted_iota(jnp.int32, (NW, BLK), 0) * BS
    oh = ((d >= lo) & (d < lo + BS)).astype(jnp.float32)
    tri = (lax.broadcasted_iota(jnp.int32, (BLK, BLK), 0)
           <= lax.broadcasted_iota(jnp.int32, (BLK, BLK), 1)).astype(jnp.float32)
    rank = jnp.dot(oh, tri, preferred_element_type=jnp.float32)
    pos = jnp.sum(oh * (rank + carry[...] + off[...] - 1.0), axis=0,
                  keepdims=True)
    p_ref[...] = jnp.reshape(pos.astype(jnp.int32), (1, 1, BLK))
    carry[...] += rank[:, BLK - 1:BLK]


_rank_call = pl.pallas_call(
    _rank_body,
    grid=(NBLK,),
    in_specs=[
        pl.BlockSpec((1, 1, BLK), lambda i: (i, 0, 0)),
        pl.BlockSpec((NW, 1), lambda i: (0, 0)),
    ],
    out_specs=[
        pl.BlockSpec((1, 1, BLK), lambda i: (i, 0, 0)),
        pl.BlockSpec((NW, 2), lambda i: (0, 0)),
    ],
    out_shape=[
        jax.ShapeDtypeStruct((NBLK, 1, BLK), jnp.int32),
        jax.ShapeDtypeStruct((NW, 2), jnp.int32),
    ],
    scratch_shapes=[
        pltpu.VMEM((NW, 1), jnp.float32),
        pltpu.VMEM((NW, 1), jnp.float32),
    ],
)


@functools.partial(
    pl.kernel,
    out_type=jax.ShapeDtypeStruct((EPAD, PW), jnp.int32),
    mesh=_mesh,
    compiler_params=_sc_params,
    scratch_types=[
        pltpu.VMEM((NCH_C, CCH), jnp.int32),
        pltpu.VMEM((NCH_C, CCH), jnp.int32),
        pltpu.VMEM((NCH_C, CCH), jnp.int32),
        pltpu.VMEM((CCH, PW), jnp.int32),
        pltpu.VMEM((CCH, PW), jnp.int32),
        pltpu.VMEM((1, CCH), jnp.int32),
        pltpu.VMEM((1, CCH), jnp.int32),
        pltpu.SemaphoreType.DMA,
        pltpu.SemaphoreType.DMA,
    ],
)
def _bucketize(src_hbm, dst_hbm, pos_hbm, pairs_hbm,
               sv, dv, pv, pb0, pb1, pi0, pi1, sem0, sem1):
    c = lax.axis_index("c")
    s = lax.axis_index("s")
    wid = s * NC + c
    pltpu.sync_copy(src_hbm.at[wid], sv)
    pltpu.sync_copy(dst_hbm.at[wid], dv)
    pltpu.sync_copy(pos_hbm.at[wid], pv)

    iota = lax.broadcasted_iota(jnp.int32, (16,), 0)
    zeros16 = iota * 0
    ones16 = zeros16 + 1

    def build(j, pb, pi):
        for k in range(CCH // 16):
            kk = 16 * k
            s16 = sv[j, pl.ds(kk, 16)]
            d16 = dv[j, pl.ds(kk, 16)]
            p16 = pv[j, pl.ds(kk, 16)]
            dl = jnp.where(d16 >= SC_ROWS, d16 - SC_ROWS, d16)
            plsc.store_scatter(pb, [kk + iota, zeros16], s16)
            plsc.store_scatter(pb, [kk + iota, ones16], dl)
            pi[0, pl.ds(kk, 16)] = p16

    for j in range(NCH_C):
        pb, pi, sem = (pb0, pi0, sem0) if j % 2 == 0 else (pb1, pi1, sem1)
        if j >= 2:
            pltpu.make_async_copy(pb, pairs_hbm.at[pi.at[0]], sem).wait()
        build(j, pb, pi)
        pltpu.async_copy(pb, pairs_hbm.at[pi.at[0]], sem)
    pltpu.make_async_copy(pb0, pairs_hbm.at[pi0.at[0]], sem0).wait()
    pltpu.make_async_copy(pb1, pairs_hbm.at[pi1.at[0]], sem1).wait()


@functools.partial(
    pl.kernel,
    out_type=jax.ShapeDtypeStruct((NC, SC_ROWS, HIDDEN), jnp.float32),
    mesh=_mesh,
    compiler_params=_sc_params,
    scratch_types=[
        pltpu.VMEM_SHARED((SC_ROWS + 8, HIDDEN), jnp.float32),
        pltpu.VMEM((2, NW), jnp.int32),
        pltpu.VMEM((LCH, PW), jnp.int32),
        pltpu.VMEM((LCH, PW), jnp.int32),
        pltpu.VMEM((LCH,), jnp.int32),
        pltpu.VMEM((1, LCH), jnp.int32),
        pltpu.VMEM((LCH,), jnp.int32),
        pltpu.VMEM((1, LCH), jnp.int32),
        pltpu.VMEM((LCH, HIDDEN), jnp.float32),
        pltpu.VMEM((LCH, HIDDEN), jnp.float32),
        pltpu.SemaphoreType.DMA,
        pltpu.SemaphoreType.DMA,
        pltpu.SemaphoreType.DMA,
        pltpu.SemaphoreType.DMA,
        pltpu.SemaphoreType.DMA,
    ],
)
def _edge_agg(h_hbm, pairs_hbm, meta_hbm, zeros_hbm, out_hbm,
              acc, meta, pb0, pb1, si0, di0, si1, di1, rows0, rows1,
              psem0, psem1, gsem0, gsem1, ssem):
    c = lax.axis_index("c")
    s = lax.axis_index("s")
    w = c * NS + s

    pltpu.sync_copy(zeros_hbm, acc.at[pl.ds(s * BS, BS)])
    pltpu.sync_copy(meta_hbm, meta)

    iota = lax.broadcasted_iota(jnp.int32, (16,), 0)
    zeros16 = iota * 0
    ones16 = zeros16 + 1

    lane = jnp.where(w >= 16, w - 16, w)

    def scal(r):
        lo16 = meta[r, pl.ds(0, 16)]
        hi16 = meta[r, pl.ds(16, 16)]
        sel = jnp.where(w >= 16, hi16, lo16)
        return jnp.sum(jnp.where(iota == lane, sel, 0))

    n = scal(0)
    off = scal(1)
    nch = (n + LCH - 1) // LCH

    def extract(j, pb, si, di):
        for k in range(LCH // 16):
            kk = 16 * k
            lane = j * LCH + kk + iota
            valid = lane < n
            s16 = plsc.load_gather(pb, [kk + iota, zeros16])
            d16 = plsc.load_gather(pb, [kk + iota, ones16])
            si[pl.ds(kk, 16)] = jnp.where(valid, s16, 0)
            di[0, pl.ds(kk, 16)] = jnp.where(valid, d16, DUMP)

    @pl.when(nch > 0)
    def _go():
        pltpu.sync_copy(pairs_hbm.at[pl.ds(off, LCH)], pb0)
        extract(0, pb0, si0, di0)

        @pl.when(nch > 1)
        def _():
            pltpu.async_copy(pairs_hbm.at[pl.ds(off + LCH, LCH)], pb1, psem1)
        pltpu.async_copy(h_hbm.at[si0], rows0, gsem0)

        def step(j, pb, si, di, rows, gsem, psem,
                 o_pb, o_si, o_di, o_rows, o_gsem, o_psem):
            @pl.when(j > 0)
            def _():
                pltpu.make_async_copy(o_rows, acc.at[o_di.at[0]], ssem).wait()

            @pl.when(j + 2 < nch)
            def _():
                pltpu.async_copy(pairs_hbm.at[pl.ds(off + (j + 2) * LCH, LCH)],
                                 pb, psem)

            @pl.when(j + 1 < nch)
            def _():
                pltpu.make_async_copy(
                    pairs_hbm.at[pl.ds(off + (j + 1) * LCH, LCH)],
                    o_pb, o_psem).wait()
                extract(j + 1, o_pb, o_si, o_di)

            @pl.when(j + 1 < nch)
            def _():
                pltpu.async_copy(h_hbm.at[o_si], o_rows, o_gsem)

            pltpu.make_async_copy(h_hbm.at[si], rows, gsem).wait()
            pltpu.async_copy(rows, acc.at[di.at[0]], ssem, add=True)

        def body(j, _):
            @pl.when(j % 2 == 0)
            def _even():
                step(j, pb0, si0, di0, rows0, gsem0, psem0,
                     pb1, si1, di1, rows1, gsem1, psem1)

            @pl.when(j % 2 == 1)
            def _odd():
                step(j, pb1, si1, di1, rows1, gsem1, psem1,
                     pb0, si0, di0, rows0, gsem0, psem0)
            return _

        lax.fori_loop(0, nch, body, None)

        @pl.when(nch % 2 == 1)
        def _():
            pltpu.make_async_copy(rows0, acc.at[di0.at[0]], ssem).wait()

        @pl.when(nch % 2 == 0)
        def _():
            pltpu.make_async_copy(rows1, acc.at[di1.at[0]], ssem).wait()

    plsc.subcore_barrier()
    pltpu.sync_copy(acc.at[pl.ds(s * BS, BS)],
                    out_hbm.at[c].at[pl.ds(s * BS, BS)])


PAIRS_PER_TILE = N_PAIRS // NW


@functools.partial(
    pl.kernel,
    out_type=[
        jax.ShapeDtypeStruct((N_PAIRS, HIDDEN), jnp.float32),
        jax.ShapeDtypeStruct((N_PAIRS, HIDDEN), jnp.float32),
    ],
    mesh=_mesh,
    compiler_params=_sc_params,
    scratch_types=[
        pltpu.VMEM((PAIRS_PER_TILE,), jnp.int32),
        pltpu.VMEM((PAIRS_PER_TILE, HIDDEN), jnp.float32),
        pltpu.SemaphoreType.DMA,
    ],
)
def _pair_gather(emb_hbm, p0_hbm, p1_hbm, ea_hbm, eb_hbm, idx_v, rows_v, sem):
    c = lax.axis_index("c")
    s = lax.axis_index("s")
    wid = s * NC + c
    base = wid * PAIRS_PER_TILE

    pltpu.sync_copy(p0_hbm.at[wid], idx_v)
    pltpu.async_copy(emb_hbm.at[idx_v], rows_v, sem).wait()
    pltpu.sync_copy(rows_v, ea_hbm.at[pl.ds(base, PAIRS_PER_TILE)])

    pltpu.sync_copy(p1_hbm.at[wid], idx_v)
    pltpu.async_copy(emb_hbm.at[idx_v], rows_v, sem).wait()
    pltpu.sync_copy(rows_v, eb_hbm.at[pl.ds(base, PAIRS_PER_TILE)])


def _layer_body(h_ref, a_ref, w1_ref, b1_ref, w2_ref, b2_ref, g_ref, be_ref, o_ref):
    agg = jnp.reshape(a_ref[...], (NC * SC_ROWS, HIDDEN))[:N_NODES]
    m = h_ref[...] + agg
    z = _dotd(m, w1_ref[...]) + b1_ref[...]
    z = jnp.maximum(z, 0.0)
    z = _dotd(z, w2_ref[...]) + b2_ref[...]
    mean = jnp.mean(z, axis=0, keepdims=True)
    cen = z - mean
    var = jnp.mean(cen * cen, axis=0, keepdims=True)
    z = cen * lax.rsqrt(var + 1e-5) * g_ref[...] + be_ref[...]
    o_ref[...] = jnp.maximum(z, 0.0)


_layer_call = pl.pallas_call(
    _layer_body,
    out_shape=jax.ShapeDtypeStruct((N_NODES, HIDDEN), jnp.float32),
)


def _heads_body(emb_ref, bi_ref, ea_ref, eb_ref,
                ncw_ref, ncb_ref, ecw_ref, ecb_ref, ccw_ref, ccb_ref,
                tcw_ref, tcb_ref, ndw_ref, ndb_ref,
                ee1a_ref, ee1b_ref, eeb1_ref, ee2_ref, eeb2_ref,
                cn1a_ref, cn1b_ref, cnb1_ref, cn2_ref, cnb2_ref,
                sp1a_ref, sp1b_ref, spb1_ref, sp2_ref, spb2_ref,
                nc_ref, ec_ref, cc_ref, tc_ref, nd_ref,
                ee_ref, cn_ref, sp_ref):
    emb = emb_ref[...]
    ohT = (lax.broadcasted_iota(jnp.int32, (N_GRAPHS, N_NODES), 0)
           == bi_ref[...]).astype(jnp.float32)
    pooled = jnp.dot(ohT, emb, preferred_element_type=jnp.float32,
                     precision=lax.Precision.HIGHEST)
    nc_ref[...] = _dotd(pooled, ncw_ref[...]) + ncb_ref[...]
    ec_ref[...] = _dotd(pooled, ecw_ref[...]) + ecb_ref[...]
    cc_ref[...] = _dotd(pooled, ccw_ref[...]) + ccb_ref[...]
    tc_ref[...] = _dotd(pooled, tcw_ref[...]) + tcb_ref[...]
    nd_ref[...] = _dotd(emb, ndw_ref[...]) + ndb_ref[...]

    ea = ea_ref[...]
    eb = eb_ref[...]

    def pair_head(w1a, w1b, b1, w2, b2, out_ref):
        h1 = _dotd(ea, w1a[...]) + _dotd(eb, w1b[...]) + b1[...]
        h1 = jnp.maximum(h1, 0.0)
        out_ref[...] = _dotd(h1, w2[...]) + b2[...]

    pair_head(ee1a_ref, ee1b_ref, eeb1_ref, ee2_ref, eeb2_ref, ee_ref)
    pair_head(cn1a_ref, cn1b_ref, cnb1_ref, cn2_ref, cnb2_ref, cn_ref)
    pair_head(sp1a_ref, sp1b_ref, spb1_ref, sp2_ref, spb2_ref, sp_ref)


_heads_call = pl.pallas_call(
    _heads_body,
    out_shape=[
        jax.ShapeDtypeStruct((N_GRAPHS, 40), jnp.float32),
        jax.ShapeDtypeStruct((N_GRAPHS, 1600), jnp.float32),
        jax.ShapeDtypeStruct((N_GRAPHS, 2), jnp.float32),
        jax.ShapeDtypeStruct((N_GRAPHS, 1), jnp.float32),
        jax.ShapeDtypeStruct((N_NODES, 40), jnp.float32),
        jax.ShapeDtypeStruct((N_PAIRS, 2), jnp.float32),
        jax.ShapeDtypeStruct((N_PAIRS, 2), jnp.float32),
        jax.ShapeDtypeStruct((N_PAIRS, 40), jnp.float32),
    ],
)


def kernel(x, edge_index, batch_index, pairs, params):
    h = x.astype(jnp.bfloat16).astype(jnp.float32)
    src = edge_index[0].astype(jnp.int32)
    dst = edge_index[1].astype(jnp.int32)
    src_r = src.reshape(NW, NCH_C, CCH)
    dst_r = dst.reshape(NW, NCH_C, CCH)
    dst_blk = dst.reshape(NBLK, 1, BLK)

    counts = _count_call(dst_blk)
    pos, meta = _rank_call(dst_blk, counts)
    pairs_sorted = _bucketize(src_r, dst_r, pos.reshape(NW, NCH_C, CCH))
    meta_t = meta.T
    zeros = jnp.zeros((BS, HIDDEN), jnp.float32)

    row = lambda v: v.reshape(1, -1)
    for i in range(NUM_LAYERS):
        agg = _edge_agg(h, pairs_sorted, meta_t, zeros)
        h = _layer_call(h, agg,
                        params['conv%d_W1' % i], row(params['conv%d_b1' % i]),
                        params['conv%d_W2' % i], row(params['conv%d_b2' % i]),
                        row(params['bn%d_gamma' % i]), row(params['bn%d_beta' % i]))

    p0 = pairs[:, 0].astype(jnp.int32).reshape(NW, PAIRS_PER_TILE)
    p1 = pairs[:, 1].astype(jnp.int32).reshape(NW, PAIRS_PER_TILE)
    ea, eb = _pair_gather(h, p0, p1)

    def split_w1(name):
        w1 = params[name + '_W1']
        return w1[:HIDDEN], w1[HIDDEN:]

    ee1a, ee1b = split_w1('edge_existence')
    cn1a, cn1b = split_w1('connectivity')
    sp1a, sp1b = split_w1('shortest_path')

    (node_count, edge_count, cycle_check, tri, node_degree,
     edge_existence, connectivity, shortest_path) = _heads_call(
        h, batch_index.astype(jnp.int32).reshape(1, N_NODES), ea, eb,
        params['node_count_W'], row(params['node_count_b']),
        params['edge_count_W'], row(params['edge_count_b']),
        params['cycle_check_W'], row(params['cycle_check_b']),
        params['triangle_count_W'], row(params['triangle_count_b']),
        params['node_degree_W'], row(params['node_degree_b']),
        ee1a, ee1b, row(params['edge_existence_b1']),
        params['edge_existence_W2'], row(params['edge_existence_b2']),
        cn1a, cn1b, row(params['connectivity_b1']),
        params['connectivity_W2'], row(params['connectivity_b2']),
        sp1a, sp1b, row(params['shortest_path_b1']),
        params['shortest_path_W2'], row(params['shortest_path_b2']),
    )
    return (node_count, edge_count, cycle_check, tri[:, 0], node_degree,
            edge_existence, connectivity, shortest_path)
